# baseline (speedup 1.0000x reference)
import jax
import jax.numpy as jnp
from jax import lax
from jax.experimental import pallas as pl
from jax.experimental.pallas import tpu as pltpu
from jax.experimental.pallas import tpu_sc as plsc
from jax._src.pallas import mpmd as _mpmd

D_MODEL = 768
VOCAB = 256
FINAL = 768
BATCH = 16384

B_SC = BATCH // 4
B_TC = BATCH - B_SC

NC = 2
NS = 16
NW = NC * NS
B_PER_W = B_SC // NW
CHUNK = 64
N_CHUNKS = B_PER_W // CHUNK

TC_BLK = 4096
N_TC_BLKS = B_TC // TC_BLK
SC_BLKS = B_SC // TC_BLK


def _mega_body(t_ref, w_ref, b_ref, x_ref, o_ref, y_ref, yb_s):
    i = pl.program_id(0)

    @pl.when(i == 0)
    def _():
        t = t_ref[...]
        h = 0.5 * t * (1.0 + lax.erf(t * 0.7071067811865476))
        y = lax.dot_general(h, w_ref[...], (((1,), (1,)), ((), ())),
                            preferred_element_type=jnp.float32,
                            precision=lax.Precision.HIGHEST)
        y = y + b_ref[...]
        y_ref[...] = y
        yb_s[...] = y.astype(jnp.bfloat16)

    @pl.when(i > 0)
    def _():
        xb = x_ref[...]
        cols = lax.broadcasted_iota(jnp.int32, (TC_BLK, VOCAB), 1)
        oh = (cols == xb[:, None]).astype(jnp.bfloat16)
        o_ref[...] = jnp.dot(oh, yb_s[...],
                             preferred_element_type=jnp.float32)


def _tc_mega(table, W, b, x):
    prev = lambda i: jnp.maximum(i - 1, 0)
    return pl.pallas_call(
        _mega_body,
        grid=(1 + N_TC_BLKS,),
        in_specs=[
            pl.BlockSpec((VOCAB, D_MODEL), lambda i: (0, 0)),
            pl.BlockSpec((FINAL, D_MODEL), lambda i: (0, 0)),
            pl.BlockSpec((1, FINAL), lambda i: (0, 0)),
            pl.BlockSpec((TC_BLK,), lambda i: (SC_BLKS + prev(i),)),
        ],
        out_specs=(
            pl.BlockSpec((TC_BLK, FINAL), lambda i: (SC_BLKS + prev(i), 0)),
            pl.BlockSpec((VOCAB, FINAL), lambda i: (0, 0)),
        ),
        out_shape=(jax.ShapeDtypeStruct((BATCH, FINAL), jnp.float32),
                   jax.ShapeDtypeStruct((VOCAB, FINAL), jnp.float32)),
        scratch_shapes=[pltpu.VMEM((VOCAB, FINAL), jnp.bfloat16)],
    )(table, W, b.reshape(1, FINAL), x)


def _gather_body(y_hbm, idx_hbm, o_in_hbm, out_hbm,
                 idx_v, buf0, buf1, sem0, sem1):
    del o_in_hbm
    wid = lax.axis_index("s") * NC + lax.axis_index("c")
    base = wid * B_PER_W
    for j in range(N_CHUNKS):
        pltpu.sync_copy(idx_hbm.at[pl.ds(base + j * CHUNK, CHUNK)],
                        idx_v.at[j])
    bufs = (buf0, buf1)
    sems = (sem0, sem1)
    copies = [None, None]
    copies[0] = pltpu.async_copy(y_hbm.at[idx_v.at[0]], bufs[0], sems[0])
    for j in range(N_CHUNKS):
        cb = j % 2
        nb = (j + 1) % 2
        if j + 1 < N_CHUNKS:
            copies[nb] = pltpu.async_copy(
                y_hbm.at[idx_v.at[j + 1]], bufs[nb], sems[nb])
        copies[cb].wait()
        pltpu.sync_copy(bufs[cb], out_hbm.at[pl.ds(base + j * CHUNK, CHUNK)])


def _sc_gather(y, x, out_tc):
    mesh = plsc.VectorSubcoreMesh(core_axis_name="c", subcore_axis_name="s")
    return _mpmd._mpmd_map(
        [(mesh, _gather_body)],
        jax.ShapeDtypeStruct((BATCH, FINAL), jnp.float32),
        input_output_aliases={2: 0},
        scratch_types=[
            pltpu.VMEM((N_CHUNKS, CHUNK), jnp.int32),
            pltpu.VMEM((CHUNK, FINAL), jnp.float32),
            pltpu.VMEM((CHUNK, FINAL), jnp.float32),
            pltpu.SemaphoreType.DMA,
            pltpu.SemaphoreType.DMA,
        ],
    )(y, x, out_tc)


def kernel(x, table, W, b):
    out_tc, y = _tc_mega(table, W, b, x)
    return _sc_gather(y, x, out_tc)


# speedup vs baseline: 1.0088x; 1.0088x over previous
import jax
import jax.numpy as jnp
from jax import lax
from jax.experimental import pallas as pl
from jax.experimental.pallas import tpu as pltpu
from jax.experimental.pallas import tpu_sc as plsc
from jax._src.pallas import mpmd as _mpmd

D_MODEL = 768
VOCAB = 256
FINAL = 768
BATCH = 16384

B_SC = BATCH // 4
B_TC = BATCH - B_SC

NC = 2
NS = 16
NW = NC * NS
B_PER_W = B_SC // NW
CHUNK = 32
N_CHUNKS = B_PER_W // CHUNK

TC_BLK = 2048
N_TC_BLKS = B_TC // TC_BLK
SC_BLKS = B_SC // TC_BLK


def _mega_body(t_ref, w_ref, b_ref, x_ref, o_ref, y_ref, yb_s):
    i = pl.program_id(0)

    @pl.when(i == 0)
    def _():
        t = t_ref[...]
        h = 0.5 * t * (1.0 + lax.erf(t * 0.7071067811865476))
        y = lax.dot_general(h, w_ref[...], (((1,), (1,)), ((), ())),
                            preferred_element_type=jnp.float32,
                            precision=lax.Precision.HIGHEST)
        y = y + b_ref[...]
        y_ref[...] = y
        yb_s[...] = y.astype(jnp.bfloat16)

    @pl.when(i > 0)
    def _():
        xb = x_ref[...]
        cols = lax.broadcasted_iota(jnp.int32, (TC_BLK, VOCAB), 1)
        oh = (cols == xb[:, None]).astype(jnp.bfloat16)
        o_ref[...] = jnp.dot(oh, yb_s[...],
                             preferred_element_type=jnp.float32)


def _tc_mega(table, W, b, x):
    prev = lambda i: jnp.maximum(i - 1, 0)
    return pl.pallas_call(
        _mega_body,
        grid=(1 + N_TC_BLKS,),
        in_specs=[
            pl.BlockSpec((VOCAB, D_MODEL), lambda i: (0, 0)),
            pl.BlockSpec((FINAL, D_MODEL), lambda i: (0, 0)),
            pl.BlockSpec((1, FINAL), lambda i: (0, 0)),
            pl.BlockSpec((TC_BLK,), lambda i: (SC_BLKS + prev(i),)),
        ],
        out_specs=(
            pl.BlockSpec((TC_BLK, FINAL), lambda i: (SC_BLKS + prev(i), 0)),
            pl.BlockSpec((VOCAB, FINAL), lambda i: (0, 0)),
        ),
        out_shape=(jax.ShapeDtypeStruct((BATCH, FINAL), jnp.float32),
                   jax.ShapeDtypeStruct((VOCAB, FINAL), jnp.float32)),
        scratch_shapes=[pltpu.VMEM((VOCAB, FINAL), jnp.bfloat16)],
    )(table, W, b.reshape(1, FINAL), x)


def _gather_body(y_hbm, idx_hbm, o_in_hbm, out_hbm, idx_v,
                 buf0, buf1, buf2, buf3, gs0, gs1, gs2, gs3,
                 ws0, ws1, ws2, ws3):
    del o_in_hbm
    wid = lax.axis_index("s") * NC + lax.axis_index("c")
    base = wid * B_PER_W
    for j in range(N_CHUNKS):
        pltpu.sync_copy(idx_hbm.at[pl.ds(base + j * CHUNK, CHUNK)],
                        idx_v.at[j])
    bufs = (buf0, buf1, buf2, buf3)
    gsems = (gs0, gs1, gs2, gs3)
    wsems = (ws0, ws1, ws2, ws3)
    gathers = [pltpu.async_copy(y_hbm.at[idx_v.at[j]], bufs[j], gsems[j])
               for j in range(N_CHUNKS)]
    writes = []
    for j in range(N_CHUNKS):
        gathers[j].wait()
        writes.append(pltpu.async_copy(
            bufs[j], out_hbm.at[pl.ds(base + j * CHUNK, CHUNK)], wsems[j]))
    for w in writes:
        w.wait()


def _sc_gather(y, x, out_tc):
    mesh = plsc.VectorSubcoreMesh(core_axis_name="c", subcore_axis_name="s")
    return _mpmd._mpmd_map(
        [(mesh, _gather_body)],
        jax.ShapeDtypeStruct((BATCH, FINAL), jnp.float32),
        input_output_aliases={2: 0},
        scratch_types=[
            pltpu.VMEM((N_CHUNKS, CHUNK), jnp.int32),
            pltpu.VMEM((CHUNK, FINAL), jnp.float32),
            pltpu.VMEM((CHUNK, FINAL), jnp.float32),
            pltpu.VMEM((CHUNK, FINAL), jnp.float32),
            pltpu.VMEM((CHUNK, FINAL), jnp.float32),
            pltpu.SemaphoreType.DMA,
            pltpu.SemaphoreType.DMA,
            pltpu.SemaphoreType.DMA,
            pltpu.SemaphoreType.DMA,
            pltpu.SemaphoreType.DMA,
            pltpu.SemaphoreType.DMA,
            pltpu.SemaphoreType.DMA,
            pltpu.SemaphoreType.DMA,
        ],
    )(y, x, out_tc)


def kernel(x, table, W, b):
    out_tc, y = _tc_mega(table, W, b, x)
    return _sc_gather(y, x, out_tc)


# speedup vs baseline: 1.0315x; 1.0226x over previous
import jax
import jax.numpy as jnp
from jax import lax
from jax.experimental import pallas as pl
from jax.experimental.pallas import tpu as pltpu
from jax.experimental.pallas import tpu_sc as plsc
from jax._src.pallas import mpmd as _mpmd

D_MODEL = 768
VOCAB = 256
FINAL = 768
BATCH = 16384

B_SC = BATCH // 4
B_TC = BATCH - B_SC

NC = 2
NS = 16
NW = NC * NS
B_PER_W = B_SC // NW
CHUNK = 64
N_CHUNKS = B_PER_W // CHUNK

TC_BLK = 2048
N_TC_BLKS = B_TC // TC_BLK
SC_BLKS = B_SC // TC_BLK


def _mega_body(t_ref, w_ref, b_ref, x_ref, o_ref, y_ref, yb_s):
    i = pl.program_id(0)

    @pl.when(i == 0)
    def _():
        t = t_ref[...]
        h = 0.5 * t * (1.0 + lax.erf(t * 0.7071067811865476))
        y = lax.dot_general(h, w_ref[...], (((1,), (1,)), ((), ())),
                            preferred_element_type=jnp.float32,
                            precision=lax.Precision.HIGHEST)
        y = y + b_ref[...]
        y_ref[...] = y
        yb_s[...] = y.astype(jnp.bfloat16)

    @pl.when(i > 0)
    def _():
        xb = x_ref[...]
        cols = lax.broadcasted_iota(jnp.int32, (TC_BLK, VOCAB), 1)
        oh = (cols == xb[:, None]).astype(jnp.bfloat16)
        o_ref[...] = jnp.dot(oh, yb_s[...],
                             preferred_element_type=jnp.float32)


def _tc_mega(table, W, b, x):
    prev = lambda i: jnp.maximum(i - 1, 0)
    return pl.pallas_call(
        _mega_body,
        grid=(1 + N_TC_BLKS,),
        in_specs=[
            pl.BlockSpec((VOCAB, D_MODEL), lambda i: (0, 0)),
            pl.BlockSpec((FINAL, D_MODEL), lambda i: (0, 0)),
            pl.BlockSpec((1, FINAL), lambda i: (0, 0)),
            pl.BlockSpec((TC_BLK,), lambda i: (SC_BLKS + prev(i),)),
        ],
        out_specs=(
            pl.BlockSpec((TC_BLK, FINAL), lambda i: (SC_BLKS + prev(i), 0)),
            pl.BlockSpec((VOCAB, FINAL), lambda i: (0, 0)),
        ),
        out_shape=(jax.ShapeDtypeStruct((BATCH, FINAL), jnp.float32),
                   jax.ShapeDtypeStruct((VOCAB, FINAL), jnp.float32)),
        scratch_shapes=[pltpu.VMEM((VOCAB, FINAL), jnp.bfloat16)],
    )(table, W, b.reshape(1, FINAL), x)


def _gather_body(y_hbm, idx_hbm, o_in_hbm, out_hbm, idx_v,
                 buf0, buf1, gs0, gs1, ws0, ws1):
    del o_in_hbm
    wid = lax.axis_index("s") * NC + lax.axis_index("c")
    base = wid * B_PER_W
    for j in range(N_CHUNKS):
        pltpu.sync_copy(idx_hbm.at[pl.ds(base + j * CHUNK, CHUNK)],
                        idx_v.at[j])
    bufs = (buf0, buf1)
    gsems = (gs0, gs1)
    wsems = (ws0, ws1)
    gathers = [pltpu.async_copy(y_hbm.at[idx_v.at[j]], bufs[j], gsems[j])
               for j in range(N_CHUNKS)]
    writes = []
    for j in range(N_CHUNKS):
        gathers[j].wait()
        writes.append(pltpu.async_copy(
            bufs[j], out_hbm.at[pl.ds(base + j * CHUNK, CHUNK)], wsems[j]))
    for w in writes:
        w.wait()


def _sc_gather(y, x, out_tc):
    mesh = plsc.VectorSubcoreMesh(core_axis_name="c", subcore_axis_name="s")
    return _mpmd._mpmd_map(
        [(mesh, _gather_body)],
        jax.ShapeDtypeStruct((BATCH, FINAL), jnp.float32),
        input_output_aliases={2: 0},
        scratch_types=[
            pltpu.VMEM((N_CHUNKS, CHUNK), jnp.int32),
            pltpu.VMEM((CHUNK, FINAL), jnp.float32),
            pltpu.VMEM((CHUNK, FINAL), jnp.float32),
            pltpu.SemaphoreType.DMA,
            pltpu.SemaphoreType.DMA,
            pltpu.SemaphoreType.DMA,
            pltpu.SemaphoreType.DMA,
        ],
    )(y, x, out_tc)


def kernel(x, table, W, b):
    out_tc, y = _tc_mega(table, W, b, x)
    return _sc_gather(y, x, out_tc)


# speedup vs baseline: 1.0407x; 1.0089x over previous
import jax
import jax.numpy as jnp
from jax import lax
from jax.experimental import pallas as pl
from jax.experimental.pallas import tpu as pltpu
from jax.experimental.pallas import tpu_sc as plsc
from jax._src.pallas import mpmd as _mpmd

D_MODEL = 768
VOCAB = 256
FINAL = 768
BATCH = 16384

B_SC = BATCH // 4
B_TC = BATCH - B_SC

NC = 2
NS = 16
NW = NC * NS
B_PER_W = B_SC // NW
CHUNK = 64
N_CHUNKS = B_PER_W // CHUNK

TC_BLK = 2048
N_TC_BLKS = B_TC // TC_BLK
SC_BLKS = B_SC // TC_BLK


def _mega_body(t_ref, w_ref, b_ref, x_ref, o_ref, y_ref, yb_s):
    i = pl.program_id(0)

    @pl.when(i == 0)
    def _():
        t = t_ref[...]
        h = 0.5 * t * (1.0 + lax.erf(t * 0.7071067811865476))
        y = lax.dot_general(h, w_ref[...], (((1,), (1,)), ((), ())),
                            preferred_element_type=jnp.float32,
                            precision=lax.Precision.HIGHEST)
        y = y + b_ref[...]
        y_ref[...] = y
        yb_s[...] = y.astype(jnp.bfloat16)

    @pl.when(i > 0)
    def _():
        xb = x_ref[...]
        cols = lax.broadcasted_iota(jnp.int32, (TC_BLK, VOCAB), 1)
        oh = (cols == xb[:, None]).astype(jnp.bfloat16)
        o_ref[...] = jnp.dot(oh, yb_s[...],
                             preferred_element_type=jnp.float32)


def _tc_mega(table, W, b, x):
    prev = lambda i: jnp.maximum(i - 1, 0)
    return pl.pallas_call(
        _mega_body,
        grid=(1 + N_TC_BLKS,),
        in_specs=[
            pl.BlockSpec((VOCAB, D_MODEL), lambda i: (0, 0)),
            pl.BlockSpec((FINAL, D_MODEL), lambda i: (0, 0)),
            pl.BlockSpec((1, FINAL), lambda i: (0, 0)),
            pl.BlockSpec((TC_BLK,), lambda i: (SC_BLKS + prev(i),)),
        ],
        out_specs=(
            pl.BlockSpec((TC_BLK, FINAL), lambda i: (SC_BLKS + prev(i), 0)),
            pl.BlockSpec((VOCAB, FINAL), lambda i: (0, 0)),
        ),
        out_shape=(jax.ShapeDtypeStruct((BATCH, FINAL), jnp.float32),
                   jax.ShapeDtypeStruct((VOCAB, FINAL), jnp.float32)),
        scratch_shapes=[pltpu.VMEM((VOCAB, FINAL), jnp.bfloat16)],
    )(table, W, b.reshape(1, FINAL), x)


def _gather_body(y_hbm, idx_hbm, o_in_hbm, out_hbm, idx_v, buf, gs):
    del o_in_hbm
    wid = lax.axis_index("s") * NC + lax.axis_index("c")
    base = wid * B_PER_W
    pltpu.sync_copy(idx_hbm.at[pl.ds(base, B_PER_W)], idx_v)
    pltpu.async_copy(y_hbm.at[idx_v], buf, gs).wait()
    pltpu.sync_copy(buf, out_hbm.at[pl.ds(base, B_PER_W)])


def _sc_gather(y, x, out_tc):
    mesh = plsc.VectorSubcoreMesh(core_axis_name="c", subcore_axis_name="s")
    return _mpmd._mpmd_map(
        [(mesh, _gather_body)],
        jax.ShapeDtypeStruct((BATCH, FINAL), jnp.float32),
        input_output_aliases={2: 0},
        scratch_types=[
            pltpu.VMEM((B_PER_W,), jnp.int32),
            pltpu.VMEM((B_PER_W, FINAL), jnp.float32),
            pltpu.SemaphoreType.DMA,
        ],
    )(y, x, out_tc)


def kernel(x, table, W, b):
    out_tc, y = _tc_mega(table, W, b, x)
    return _sc_gather(y, x, out_tc)


# speedup vs baseline: 1.0862x; 1.0437x over previous
import jax
import jax.numpy as jnp
from jax import lax
from jax.experimental import pallas as pl
from jax.experimental.pallas import tpu as pltpu
from jax.experimental.pallas import tpu_sc as plsc
from jax._src.pallas import mpmd as _mpmd

D_MODEL = 768
VOCAB = 256
FINAL = 768
BATCH = 16384

B_SC = BATCH // 4
B_TC = BATCH - B_SC

NC = 2
NS = 16
NW = NC * NS
B_PER_W = B_SC // NW
CHUNK = 64
N_CHUNKS = B_PER_W // CHUNK

TC_BLK = 2048
N_TC_BLKS = B_TC // TC_BLK
SC_BLKS = B_SC // TC_BLK


def _mega_body(t_ref, w_ref, b_ref, x_ref, o_ref, y_ref, yb_s):
    i = pl.program_id(0)

    @pl.when(i == 0)
    def _():
        t = t_ref[...]
        h = 0.5 * t * (1.0 + lax.erf(t * 0.7071067811865476))
        y = lax.dot_general(h, w_ref[...], (((1,), (1,)), ((), ())),
                            preferred_element_type=jnp.float32)
        y = y + b_ref[...]
        y_ref[...] = y
        yb_s[...] = y.astype(jnp.bfloat16)

    @pl.when(i > 0)
    def _():
        xb = x_ref[...]
        cols = lax.broadcasted_iota(jnp.int32, (TC_BLK, VOCAB), 1)
        oh = (cols == xb[:, None]).astype(jnp.bfloat16)
        o_ref[...] = jnp.dot(oh, yb_s[...],
                             preferred_element_type=jnp.float32)


def _tc_mega(table, W, b, x):
    prev = lambda i: jnp.maximum(i - 1, 0)
    return pl.pallas_call(
        _mega_body,
        grid=(1 + N_TC_BLKS,),
        in_specs=[
            pl.BlockSpec((VOCAB, D_MODEL), lambda i: (0, 0)),
            pl.BlockSpec((FINAL, D_MODEL), lambda i: (0, 0)),
            pl.BlockSpec((1, FINAL), lambda i: (0, 0)),
            pl.BlockSpec((TC_BLK,), lambda i: (SC_BLKS + prev(i),)),
        ],
        out_specs=(
            pl.BlockSpec((TC_BLK, FINAL), lambda i: (SC_BLKS + prev(i), 0)),
            pl.BlockSpec((VOCAB, FINAL), lambda i: (0, 0)),
        ),
        out_shape=(jax.ShapeDtypeStruct((BATCH, FINAL), jnp.float32),
                   jax.ShapeDtypeStruct((VOCAB, FINAL), jnp.float32)),
        scratch_shapes=[pltpu.VMEM((VOCAB, FINAL), jnp.bfloat16)],
    )(table, W, b.reshape(1, FINAL), x)


def _gather_body(y_hbm, idx_hbm, o_in_hbm, out_hbm, idx_v, buf, gs):
    del o_in_hbm
    wid = lax.axis_index("s") * NC + lax.axis_index("c")
    base = wid * B_PER_W
    pltpu.sync_copy(idx_hbm.at[pl.ds(base, B_PER_W)], idx_v)
    pltpu.async_copy(y_hbm.at[idx_v], buf, gs).wait()
    pltpu.sync_copy(buf, out_hbm.at[pl.ds(base, B_PER_W)])


def _sc_gather(y, x, out_tc):
    mesh = plsc.VectorSubcoreMesh(core_axis_name="c", subcore_axis_name="s")
    return _mpmd._mpmd_map(
        [(mesh, _gather_body)],
        jax.ShapeDtypeStruct((BATCH, FINAL), jnp.float32),
        input_output_aliases={2: 0},
        scratch_types=[
            pltpu.VMEM((B_PER_W,), jnp.int32),
            pltpu.VMEM((B_PER_W, FINAL), jnp.float32),
            pltpu.SemaphoreType.DMA,
        ],
    )(y, x, out_tc)


def kernel(x, table, W, b):
    out_tc, y = _tc_mega(table, W, b, x)
    return _sc_gather(y, x, out_tc)
